# baseline (speedup 1.0000x reference)
import functools

import jax
import jax.numpy as jnp
from jax import lax
from jax.experimental import pallas as pl
from jax.experimental.pallas import tpu as pltpu
from jax.experimental.pallas import tpu_sc as plsc

_NA = 10000
_EA = 160000
_EB = 320000
_D = 128
_H = 256

_NC = 2
_NS = 16
_NW = _NC * _NS
_CK = 128

_SP1 = 10240
_SP2C = 8192
_NCH2 = 20
_SP2 = _SP2C * _NCH2


def _mesh():
    return plsc.VectorSubcoreMesh(core_axis_name="c", subcore_axis_name="s")


def _zero_zb(zb):
    def zrow(i, carry):
        for v in range(_D // 16):
            zb[i, pl.ds(v * 16, 16)] = jnp.zeros((16,), jnp.float32)
        return carry

    lax.fori_loop(0, _CK, zrow, 0)


def _gather2_body(nch, table, idx0, idx1, out0, out1,
                  iv0, iv1, buf0, buf1, sem0, sem1):
    wid = lax.axis_index("s") * _NC + lax.axis_index("c")
    nj = (nch + _NW - 1) // _NW

    def step(j, carry):
        c = wid + j * _NW

        @pl.when(c < nch)
        def _():
            r0 = c * _CK
            i0 = pltpu.async_copy(idx0.at[pl.ds(r0, _CK)], iv0, sem0)
            i1 = pltpu.async_copy(idx1.at[pl.ds(r0, _CK)], iv1, sem0)
            i0.wait()
            i1.wait()
            cp0 = pltpu.async_copy(table.at[iv0], buf0, sem1)
            cp1 = pltpu.async_copy(table.at[iv1], buf1, sem1)
            cp0.wait()
            cp1.wait()
            w0 = pltpu.async_copy(buf0, out0.at[pl.ds(r0, _CK)], sem0)
            w1 = pltpu.async_copy(buf1, out1.at[pl.ds(r0, _CK)], sem0)
            w0.wait()
            w1.wait()

        return carry

    lax.fori_loop(0, nj, step, 0)


def _gather2(table, idx0, idx1):
    e = idx0.shape[0]
    nch = e // _CK
    out = jax.ShapeDtypeStruct((e, _D), jnp.float32)
    f = pl.kernel(
        functools.partial(_gather2_body, nch),
        out_type=(out, out),
        mesh=_mesh(),
        scratch_types=[
            pltpu.VMEM((_CK,), jnp.int32),
            pltpu.VMEM((_CK,), jnp.int32),
            pltpu.VMEM((_CK, _D), jnp.float32),
            pltpu.VMEM((_CK, _D), jnp.float32),
            pltpu.SemaphoreType.DMA,
            pltpu.SemaphoreType.DMA,
        ],
    )
    return f(table, idx0, idx1)


def _gather1_body(nch, table, idx0, out0, iv0, buf0, sem0, sem1):
    wid = lax.axis_index("s") * _NC + lax.axis_index("c")
    nj = (nch + _NW - 1) // _NW

    def step(j, carry):
        c = wid + j * _NW

        @pl.when(c < nch)
        def _():
            r0 = c * _CK
            pltpu.async_copy(idx0.at[pl.ds(r0, _CK)], iv0, sem0).wait()
            pltpu.async_copy(table.at[iv0], buf0, sem1).wait()
            pltpu.async_copy(buf0, out0.at[pl.ds(r0, _CK)], sem0).wait()

        return carry

    lax.fori_loop(0, nj, step, 0)


def _gather1(table, idx0):
    e = idx0.shape[0]
    nch = e // _CK
    out = jax.ShapeDtypeStruct((e, _D), jnp.float32)
    f = pl.kernel(
        functools.partial(_gather1_body, nch),
        out_type=out,
        mesh=_mesh(),
        scratch_types=[
            pltpu.VMEM((_CK,), jnp.int32),
            pltpu.VMEM((_CK, _D), jnp.float32),
            pltpu.SemaphoreType.DMA,
            pltpu.SemaphoreType.DMA,
        ],
    )
    return f(table, idx0)


def _segsum1_body(m, dstv, hout, acc, buf, lidx, zb):
    cid = lax.axis_index("c")
    sid = lax.axis_index("s")
    _zero_zb(zb)
    rpt = _SP1 // _NS
    for r in range(rpt // _CK):
        pltpu.sync_copy(zb, acc.at[pl.ds(sid * rpt + r * _CK, _CK)])
    plsc.subcore_barrier()

    nch = _EA // _CK
    nch_core = nch // _NC
    nj = (nch_core + _NS - 1) // _NS

    def step(j, carry):
        lc = sid + j * _NS

        @pl.when(lc < nch_core)
        def _():
            r0 = (cid * nch_core + lc) * _CK
            pltpu.sync_copy(dstv.at[pl.ds(r0, _CK)], lidx)
            pltpu.sync_copy(m.at[pl.ds(r0, _CK)], buf)
            pltpu.sync_copy(buf, acc.at[lidx], add=True)

        return carry

    lax.fori_loop(0, nj, step, 0)
    plsc.subcore_barrier()
    for r in range(rpt // _CK):
        rr = sid * rpt + r * _CK
        pltpu.sync_copy(acc.at[pl.ds(rr, _CK)], buf)
        pltpu.sync_copy(buf, hout.at[cid, pl.ds(rr, _CK)])


def _segsum1(m, dstv):
    f = pl.kernel(
        _segsum1_body,
        out_type=jax.ShapeDtypeStruct((_NC, _SP1, _D), jnp.float32),
        mesh=_mesh(),
        scratch_types=[
            pltpu.VMEM_SHARED((_SP1, _D), jnp.float32),
            pltpu.VMEM((_CK, _D), jnp.float32),
            pltpu.VMEM((_CK,), jnp.int32),
            pltpu.VMEM((_CK, _D), jnp.float32),
        ],
    )
    return f(m, dstv)


_GB = 128
_GPAD = 2560
_NBK = 20


def _hist_body(dst_ref, out_ref, acc_ref):
    i = pl.program_id(0)

    @pl.when(i == 0)
    def _():
        acc_ref[...] = jnp.zeros_like(acc_ref)

    bucket = lax.shift_right_logical(dst_ref[...], 13)
    iot = lax.broadcasted_iota(jnp.int32, (1, _CK), 1)
    add = jnp.zeros((1, _CK), jnp.float32)
    for c in range(_NBK):
        cnt = jnp.sum((bucket == c).astype(jnp.float32))
        add = add + jnp.where(iot == c, cnt, 0.0)
    acc_ref[...] += add

    @pl.when(i == pl.num_programs(0) - 1)
    def _():
        out_ref[...] = acc_ref[...]


def _tc_hist(dst2d):
    g = dst2d.shape[0]
    return pl.pallas_call(
        _hist_body,
        grid=(g // _GB,),
        in_specs=[pl.BlockSpec((_GB, _CK), lambda i: (i, 0))],
        out_specs=pl.BlockSpec((1, _CK), lambda i: (0, 0)),
        out_shape=jax.ShapeDtypeStruct((1, _CK), jnp.float32),
        scratch_shapes=[pltpu.VMEM((1, _CK), jnp.float32)],
        compiler_params=pltpu.CompilerParams(
            dimension_semantics=("arbitrary",)),
    )(dst2d)


def _pos_body(dst_ref, cnt_ref, pos_ref, off_ref, carry_ref):
    i = pl.program_id(0)

    @pl.when(i == 0)
    def _():
        carry_ref[...] = jnp.zeros_like(carry_ref)

    offs = []
    racc = jnp.float32(0.0)
    for c in range(_NBK + 1):
        offs.append(racc)
        racc = racc + cnt_ref[0, c]

    rg = lax.broadcasted_iota(jnp.int32, (_GB, _GB), 0)
    cg = lax.broadcasted_iota(jnp.int32, (_GB, _GB), 1)
    trig_strict = (rg > cg).astype(jnp.float32)
    rl = lax.broadcasted_iota(jnp.int32, (_CK, _CK), 0)
    cl = lax.broadcasted_iota(jnp.int32, (_CK, _CK), 1)
    tril_incl = (rl <= cl).astype(jnp.float32)

    bucket = lax.shift_right_logical(dst_ref[...], 13)
    iot = lax.broadcasted_iota(jnp.int32, (1, _CK), 1)
    carr = carry_ref[...]
    cadd = jnp.zeros((1, _CK), jnp.float32)
    pos = jnp.zeros((_GB, _CK), jnp.float32)
    for cc in range(_NBK):
        mask = (bucket == cc).astype(jnp.float32)
        lane_pre = jnp.dot(mask, tril_incl,
                           preferred_element_type=jnp.float32)
        rowtot = lane_pre[:, _CK - 1:_CK]
        row_pre = jnp.dot(trig_strict, rowtot,
                          preferred_element_type=jnp.float32)
        slot = offs[cc] + carr[0, cc] + row_pre + lane_pre - 1.0
        pos += mask * slot
        cadd = cadd + jnp.where(iot == cc, jnp.sum(rowtot), 0.0)
    carry_ref[...] = carr + cadd

    pos_ref[...] = pos.astype(jnp.int32)

    @pl.when(i == pl.num_programs(0) - 1)
    def _():
        iot2 = lax.broadcasted_iota(jnp.int32, (1, _CK), 1)
        ofr = jnp.zeros((1, _CK), jnp.float32)
        for c in range(_NBK + 1):
            ofr = ofr + jnp.where(iot2 == c, offs[c], 0.0)
        off_ref[...] = ofr.astype(jnp.int32)


def _tc_pos(dst2d, counts):
    g = dst2d.shape[0]
    return pl.pallas_call(
        _pos_body,
        grid=(g // _GB,),
        in_specs=[pl.BlockSpec((_GB, _CK), lambda i: (i, 0)),
                  pl.BlockSpec((1, _CK), lambda i: (0, 0))],
        out_specs=[pl.BlockSpec((_GB, _CK), lambda i: (i, 0)),
                   pl.BlockSpec((1, _CK), lambda i: (0, 0))],
        out_shape=[jax.ShapeDtypeStruct((g, _CK), jnp.int32),
                   jax.ShapeDtypeStruct((1, _CK), jnp.int32)],
        scratch_shapes=[pltpu.VMEM((1, _CK), jnp.float32)],
        compiler_params=pltpu.CompilerParams(
            dimension_semantics=("arbitrary",)),
    )(dst2d, counts)


_STG = 321536


def _segsum2_body(m, posv, dstv, offh, hout,
                  acc, shids, shrel, offv, idrow, relrow, mbuf, fbuf, sem):
    cid = lax.axis_index("c")
    sid = lax.axis_index("s")
    rpt = _SP2C // _NS
    nblk = _EB // _CK
    iota16 = lax.iota(jnp.int32, 16)
    pltpu.sync_copy(offh, offv.at[pl.ds(0, 32)])
    zf = jnp.zeros((16,), jnp.float32)
    zi = jnp.zeros((16,), jnp.int32)

    def frow(i, carry):
        fbuf[pl.ds(i * 16, 16)] = zi
        return carry

    lax.fori_loop(0, 128, frow, 0)

    def fill(j, carry):
        p = (sid + j * _NS) * 2048

        @pl.when(p < _STG)
        def _():
            pltpu.sync_copy(fbuf, shrel.at[pl.ds(p, 2048)])
            pltpu.sync_copy(fbuf, shids.at[pl.ds(p, 2048)])

        return carry

    lax.fori_loop(0, (_STG // 2048 + _NS - 1) // _NS, fill, 0)
    plsc.subcore_barrier()

    def scat(j, carry):
        b = sid + j * _NS

        @pl.when(b < nblk)
        def _():
            r0 = b * _CK
            i0 = pltpu.async_copy(posv.at[pl.ds(r0, _CK)], idrow, sem)
            i1 = pltpu.async_copy(dstv.at[pl.ds(r0, _CK)], relrow, sem)
            i0.wait()
            i1.wait()
            for v in range(_CK // 16):
                pv = idrow[pl.ds(v * 16, 16)]
                dv = relrow[pl.ds(v * 16, 16)]
                own = lax.shift_right_logical(dv, 13) % 2 == cid
                pv = jnp.minimum(jnp.maximum(pv, 0), _EB - 1)
                idrow[pl.ds(v * 16, 16)] = jnp.where(own, pv, _EB)
                relrow[pl.ds(v * 16, 16)] = dv + 1
                fbuf[pl.ds(v * 16, 16)] = (r0 + v * 16) + iota16
            pltpu.sync_copy(fbuf.at[pl.ds(0, _CK)], shids.at[idrow],
                            add=True)
            pltpu.sync_copy(relrow, shrel.at[idrow], add=True)

        return carry

    lax.fori_loop(0, (nblk + _NS - 1) // _NS, scat, 0)
    plsc.subcore_barrier()

    def chunk_loop(kc, carry):
        t = cid + kc * _NC
        lo = t * _SP2C
        o0 = offv[pl.ds(t, 16)][0]
        o1 = offv[pl.ds(t + 1, 16)][0]
        b0 = jnp.minimum(jnp.maximum(lax.shift_right_logical(o0, 7), 0),
                         nblk)
        b1 = jnp.minimum(jnp.maximum(lax.shift_right_logical(o1 + 127, 7),
                                     0), nblk)

        def zrow(i, zcarry):
            for v in range(_D // 16):
                mbuf[i, pl.ds(v * 16, 16)] = zf
            return zcarry

        lax.fori_loop(0, _CK, zrow, 0)
        for r in range(rpt // _CK):
            pltpu.sync_copy(mbuf, acc.at[pl.ds(sid * rpt + r * _CK, _CK)])
        plsc.subcore_barrier()

        def gat(j, gcarry):
            b = b0 + sid + j * _NS

            @pl.when(b < b1)
            def _():
                r0 = b * _CK
                pltpu.sync_copy(shids.at[pl.ds(r0, _CK)], idrow)
                pltpu.sync_copy(shrel.at[pl.ds(r0, _CK)], relrow)
                for v in range(_D // 16):
                    rr = relrow[pl.ds(v * 16, 16)] - (lo + 1)
                    ok = (rr >= 0) & (rr < _SP2C)
                    relrow[pl.ds(v * 16, 16)] = jnp.where(
                        ok, rr, jnp.full((16,), _SP2C, jnp.int32))
                    iv = idrow[pl.ds(v * 16, 16)]
                    idrow[pl.ds(v * 16, 16)] = jnp.minimum(
                        jnp.maximum(iv, 0), _EB - 1)
                pltpu.async_copy(m.at[idrow], mbuf, sem).wait()
                pltpu.sync_copy(mbuf, acc.at[relrow], add=True)

            return gcarry

        lax.fori_loop(0, (b1 - b0 + _NS - 1) // _NS, gat, 0)
        plsc.subcore_barrier()
        for r in range(rpt // _CK):
            rr = sid * rpt + r * _CK
            pltpu.sync_copy(acc.at[pl.ds(rr, _CK)], mbuf)
            pltpu.sync_copy(mbuf, hout.at[pl.ds(lo + rr, _CK)])
        plsc.subcore_barrier()
        return carry

    lax.fori_loop(0, _NCH2 // _NC, chunk_loop, 0)


def _segsum2(m, dstv):
    pad = _GPAD * _CK - _EB
    dstp = jnp.concatenate(
        [dstv, jnp.full((pad,), _NBK * _SP2C, jnp.int32)])
    dst2d = dstp.reshape(_GPAD, _CK)
    counts = _tc_hist(dst2d)
    pos2d, off = _tc_pos(dst2d, counts)
    f = pl.kernel(
        _segsum2_body,
        out_type=jax.ShapeDtypeStruct((_SP2, _D), jnp.float32),
        mesh=_mesh(),
        scratch_types=[
            pltpu.VMEM_SHARED((_SP2C + 8, _D), jnp.float32),
            pltpu.VMEM_SHARED((_STG,), jnp.int32),
            pltpu.VMEM_SHARED((_STG,), jnp.int32),
            pltpu.VMEM((64,), jnp.int32),
            pltpu.VMEM((_CK,), jnp.int32),
            pltpu.VMEM((_CK,), jnp.int32),
            pltpu.VMEM((_CK, _D), jnp.float32),
            pltpu.VMEM((2048,), jnp.int32),
            pltpu.SemaphoreType.DMA,
        ],
    )
    return f(m, pos2d.reshape(_GPAD * _CK)[:_EB], dstv,
             off.reshape(_CK)[:32])


_BE = 1280


def _sigmoid(x):
    return 1.0 / (1.0 + jnp.exp(-x))


def _silu(x):
    return x * _sigmoid(x)


def _mlp_body(n_in, n_mult, residual, *refs):
    xs = refs[:n_in]
    i = n_in
    mults = refs[i:i + n_mult]
    i += n_mult
    res = refs[i] if residual else None
    i += 1 if residual else 0
    gw1, gb1, gw2, gb2, ow1, ob1, ow2, ob2, out = refs[i:i + 9]

    xcat = jnp.concatenate([x[...] for x in xs],
                           axis=1).astype(jnp.bfloat16)
    ag = jnp.dot(xcat, gw1[...].astype(jnp.bfloat16),
                 preferred_element_type=jnp.float32)
    ao = jnp.dot(xcat, ow1[...].astype(jnp.bfloat16),
                 preferred_element_type=jnp.float32)
    hg = _silu(ag + gb1[...]).astype(jnp.bfloat16)
    ho = _silu(ao + ob1[...]).astype(jnp.bfloat16)
    g = _sigmoid(jnp.dot(hg, gw2[...].astype(jnp.bfloat16),
                         preferred_element_type=jnp.float32) + gb2[...])
    o = _silu(jnp.dot(ho, ow2[...].astype(jnp.bfloat16),
                      preferred_element_type=jnp.float32) + ob2[...])
    y = o * g
    for mr in mults:
        y = y * mr[...]
    if residual:
        y = y + res[...]
    out[...] = y


def _mlp(xs, mults, res, p):
    e = xs[0].shape[0]
    n_in = len(xs)
    grid = (e // _BE,)
    row = pl.BlockSpec((_BE, _D), lambda i: (i, 0))
    w1s = pl.BlockSpec((n_in * _D, _H), lambda i: (0, 0))
    b1s = pl.BlockSpec((1, _H), lambda i: (0, 0))
    w2s = pl.BlockSpec((_H, _D), lambda i: (0, 0))
    b2s = pl.BlockSpec((1, _D), lambda i: (0, 0))
    n_row = n_in + len(mults) + (1 if res is not None else 0)
    in_specs = [row] * n_row + [w1s, b1s, w2s, b2s, w1s, b1s, w2s, b2s]
    gw1 = p['gw1']
    ow1 = p['ow1']
    args = ([*xs, *mults] + ([res] if res is not None else [])
            + [gw1, p['gb1'].reshape(1, _H), p['gw2'], p['gb2'].reshape(1, _D),
               ow1, p['ob1'].reshape(1, _H), p['ow2'], p['ob2'].reshape(1, _D)])
    return pl.pallas_call(
        functools.partial(_mlp_body, n_in, len(mults), res is not None),
        grid=grid,
        in_specs=in_specs,
        out_specs=row,
        out_shape=jax.ShapeDtypeStruct((e, _D), jnp.float32),
        compiler_params=pltpu.CompilerParams(
            dimension_semantics=("arbitrary",)),
    )(*args)


def _lin_body(nh, *refs):
    feat = refs[0]
    hs = refs[1:1 + nh]
    w, b, out = refs[1 + nh:1 + nh + 3]
    h = hs[0][...]
    for k in range(1, nh):
        h = h + hs[k][...]
    out[...] = (feat[...]
                + jnp.dot(h, w[...], preferred_element_type=jnp.float32)
                + b[...])


def _lin(feat, hs, w, b, be):
    e = feat.shape[0]
    grid = (e // be,)
    row = pl.BlockSpec((be, _D), lambda i: (i, 0))
    ws = pl.BlockSpec((_D, _D), lambda i: (0, 0))
    bs = pl.BlockSpec((1, _D), lambda i: (0, 0))
    return pl.pallas_call(
        functools.partial(_lin_body, len(hs)),
        grid=grid,
        in_specs=[row] * (1 + len(hs)) + [ws, bs],
        out_specs=row,
        out_shape=jax.ShapeDtypeStruct((e, _D), jnp.float32),
        compiler_params=pltpu.CompilerParams(
            dimension_semantics=("arbitrary",)),
    )(feat, *hs, w, b.reshape(1, _D))


def kernel(atom_feat, bond_feat, angle_feat, atom_edge_index, bond_edge_index,
           angle_index, atom_bond_weight, bond_node_weight, params):
    src_a = atom_edge_index[0]
    dst_a = atom_edge_index[1]
    src_b = bond_edge_index[0]
    dst_b = bond_edge_index[1]
    vertex = angle_index[:, 1]

    g1s, g1d = _gather2(atom_feat, src_a, dst_a)
    bfs, bfd = _gather2(bond_feat, src_b, dst_b)
    ws, wd = _gather2(bond_node_weight, src_b, dst_b)
    m1 = _mlp([g1s, g1d, bond_feat], [atom_bond_weight], None,
              params['atom_conv'])
    hparts = _segsum1(m1, dst_a)
    atom_out = _lin(atom_feat, [hparts[0, :_NA], hparts[1, :_NA]],
                    params['atom_lin']['w'], params['atom_lin']['b'], 1000)

    vf = _gather1(atom_out, vertex)
    m2 = _mlp([bfs, bfd, angle_feat, vf], [ws, wd], None, params['bond_conv'])
    h2 = _segsum2(m2, dst_b)
    bond_out = _lin(bond_feat, [h2[:_EA]],
                    params['bond_lin']['w'], params['bond_lin']['b'], _BE)

    g3s, g3d = _gather2(bond_out, src_b, dst_b)
    angle_out = _mlp([g3s, g3d, angle_feat, vf], [], angle_feat,
                     params['angle_update'])

    return (atom_out, bond_out, angle_out)


# speedup vs baseline: 1.0682x; 1.0682x over previous
import functools

import jax
import jax.numpy as jnp
from jax import lax
from jax.experimental import pallas as pl
from jax.experimental.pallas import tpu as pltpu
from jax.experimental.pallas import tpu_sc as plsc

_NA = 10000
_EA = 160000
_EB = 320000
_D = 128
_H = 256

_NC = 2
_NS = 16
_NW = _NC * _NS
_CK = 128

_SP1 = 10240
_SP2C = 8192
_NCH2 = 20
_SP2 = _SP2C * _NCH2


def _mesh():
    return plsc.VectorSubcoreMesh(core_axis_name="c", subcore_axis_name="s")


def _zero_zb(zb):
    def zrow(i, carry):
        for v in range(_D // 16):
            zb[i, pl.ds(v * 16, 16)] = jnp.zeros((16,), jnp.float32)
        return carry

    lax.fori_loop(0, _CK, zrow, 0)


def _gather2_body(nch, table, idx0, idx1, out0, out1,
                  ivA0, ivA1, ivB0, ivB1, bA0, bA1, bB0, bB1,
                  sIA, sIB, sGA, sGB, sWA, sWB):
    wid = lax.axis_index("s") * _NC + lax.axis_index("c")
    nj = (nch + _NW - 1) // _NW
    npair = (nj + 1) // 2

    def pair(k, carry):
        c0 = wid + (2 * k) * _NW
        c1 = wid + (2 * k + 1) * _NW
        r0 = c0 * _CK
        r1 = c1 * _CK

        @pl.when(c0 < nch)
        def _():
            pltpu.async_copy(idx0.at[pl.ds(r0, _CK)], ivA0, sIA)
            pltpu.async_copy(idx1.at[pl.ds(r0, _CK)], ivA1, sIA)

        @pl.when(c1 < nch)
        def _():
            pltpu.async_copy(idx0.at[pl.ds(r1, _CK)], ivB0, sIB)
            pltpu.async_copy(idx1.at[pl.ds(r1, _CK)], ivB1, sIB)

        @pl.when(c0 < nch)
        def _():
            pltpu.make_async_copy(idx0.at[pl.ds(r0, _CK)], ivA0, sIA).wait()
            pltpu.make_async_copy(idx1.at[pl.ds(r0, _CK)], ivA1, sIA).wait()
            pltpu.async_copy(table.at[ivA0], bA0, sGA)
            pltpu.async_copy(table.at[ivA1], bA1, sGA)

        @pl.when(c1 < nch)
        def _():
            pltpu.make_async_copy(idx0.at[pl.ds(r1, _CK)], ivB0, sIB).wait()
            pltpu.make_async_copy(idx1.at[pl.ds(r1, _CK)], ivB1, sIB).wait()
            pltpu.async_copy(table.at[ivB0], bB0, sGB)
            pltpu.async_copy(table.at[ivB1], bB1, sGB)

        @pl.when(c0 < nch)
        def _():
            pltpu.make_async_copy(table.at[ivA0], bA0, sGA).wait()
            pltpu.make_async_copy(table.at[ivA1], bA1, sGA).wait()
            pltpu.async_copy(bA0, out0.at[pl.ds(r0, _CK)], sWA)
            pltpu.async_copy(bA1, out1.at[pl.ds(r0, _CK)], sWA)

        @pl.when(c1 < nch)
        def _():
            pltpu.make_async_copy(table.at[ivB0], bB0, sGB).wait()
            pltpu.make_async_copy(table.at[ivB1], bB1, sGB).wait()
            pltpu.async_copy(bB0, out0.at[pl.ds(r1, _CK)], sWB)
            pltpu.async_copy(bB1, out1.at[pl.ds(r1, _CK)], sWB)

        @pl.when(c0 < nch)
        def _():
            pltpu.make_async_copy(bA0, out0.at[pl.ds(r0, _CK)], sWA).wait()
            pltpu.make_async_copy(bA1, out1.at[pl.ds(r0, _CK)], sWA).wait()

        @pl.when(c1 < nch)
        def _():
            pltpu.make_async_copy(bB0, out0.at[pl.ds(r1, _CK)], sWB).wait()
            pltpu.make_async_copy(bB1, out1.at[pl.ds(r1, _CK)], sWB).wait()

        return carry

    lax.fori_loop(0, npair, pair, 0)


def _gather2(table, idx0, idx1):
    e = idx0.shape[0]
    nch = e // _CK
    out = jax.ShapeDtypeStruct((e, _D), jnp.float32)
    f = pl.kernel(
        functools.partial(_gather2_body, nch),
        out_type=(out, out),
        mesh=_mesh(),
        scratch_types=[pltpu.VMEM((_CK,), jnp.int32)] * 4
        + [pltpu.VMEM((_CK, _D), jnp.float32)] * 4
        + [pltpu.SemaphoreType.DMA] * 6,
    )
    return f(table, idx0, idx1)


def _gather1_body(nch, table, idx0, out0, iv0, buf0, sem0, sem1):
    wid = lax.axis_index("s") * _NC + lax.axis_index("c")
    nj = (nch + _NW - 1) // _NW

    def step(j, carry):
        c = wid + j * _NW

        @pl.when(c < nch)
        def _():
            r0 = c * _CK
            pltpu.async_copy(idx0.at[pl.ds(r0, _CK)], iv0, sem0).wait()
            pltpu.async_copy(table.at[iv0], buf0, sem1).wait()
            pltpu.async_copy(buf0, out0.at[pl.ds(r0, _CK)], sem0).wait()

        return carry

    lax.fori_loop(0, nj, step, 0)


def _gather1(table, idx0):
    e = idx0.shape[0]
    nch = e // _CK
    out = jax.ShapeDtypeStruct((e, _D), jnp.float32)
    f = pl.kernel(
        functools.partial(_gather1_body, nch),
        out_type=out,
        mesh=_mesh(),
        scratch_types=[
            pltpu.VMEM((_CK,), jnp.int32),
            pltpu.VMEM((_CK, _D), jnp.float32),
            pltpu.SemaphoreType.DMA,
            pltpu.SemaphoreType.DMA,
        ],
    )
    return f(table, idx0)


def _segsum1_body(m, dstv, hout, acc, buf, lidx, zb):
    cid = lax.axis_index("c")
    sid = lax.axis_index("s")
    _zero_zb(zb)
    rpt = _SP1 // _NS
    for r in range(rpt // _CK):
        pltpu.sync_copy(zb, acc.at[pl.ds(sid * rpt + r * _CK, _CK)])
    plsc.subcore_barrier()

    nch = _EA // _CK
    nch_core = nch // _NC
    nj = (nch_core + _NS - 1) // _NS

    def step(j, carry):
        lc = sid + j * _NS

        @pl.when(lc < nch_core)
        def _():
            r0 = (cid * nch_core + lc) * _CK
            pltpu.sync_copy(dstv.at[pl.ds(r0, _CK)], lidx)
            pltpu.sync_copy(m.at[pl.ds(r0, _CK)], buf)
            pltpu.sync_copy(buf, acc.at[lidx], add=True)

        return carry

    lax.fori_loop(0, nj, step, 0)
    plsc.subcore_barrier()
    for r in range(rpt // _CK):
        rr = sid * rpt + r * _CK
        pltpu.sync_copy(acc.at[pl.ds(rr, _CK)], buf)
        pltpu.sync_copy(buf, hout.at[cid, pl.ds(rr, _CK)])


def _segsum1(m, dstv):
    f = pl.kernel(
        _segsum1_body,
        out_type=jax.ShapeDtypeStruct((_NC, _SP1, _D), jnp.float32),
        mesh=_mesh(),
        scratch_types=[
            pltpu.VMEM_SHARED((_SP1, _D), jnp.float32),
            pltpu.VMEM((_CK, _D), jnp.float32),
            pltpu.VMEM((_CK,), jnp.int32),
            pltpu.VMEM((_CK, _D), jnp.float32),
        ],
    )
    return f(m, dstv)


_GB = 128
_GPAD = 2560
_NBK = 20


def _hist_body(dst_ref, out_ref, acc_ref):
    i = pl.program_id(0)

    @pl.when(i == 0)
    def _():
        acc_ref[...] = jnp.zeros_like(acc_ref)

    bucket = lax.shift_right_logical(dst_ref[...], 13)
    iot = lax.broadcasted_iota(jnp.int32, (1, _CK), 1)
    add = jnp.zeros((1, _CK), jnp.float32)
    for c in range(_NBK):
        cnt = jnp.sum((bucket == c).astype(jnp.float32))
        add = add + jnp.where(iot == c, cnt, 0.0)
    acc_ref[...] += add

    @pl.when(i == pl.num_programs(0) - 1)
    def _():
        out_ref[...] = acc_ref[...]


def _tc_hist(dst2d):
    g = dst2d.shape[0]
    return pl.pallas_call(
        _hist_body,
        grid=(g // _GB,),
        in_specs=[pl.BlockSpec((_GB, _CK), lambda i: (i, 0))],
        out_specs=pl.BlockSpec((1, _CK), lambda i: (0, 0)),
        out_shape=jax.ShapeDtypeStruct((1, _CK), jnp.float32),
        scratch_shapes=[pltpu.VMEM((1, _CK), jnp.float32)],
        compiler_params=pltpu.CompilerParams(
            dimension_semantics=("arbitrary",)),
    )(dst2d)


def _pos_body(dst_ref, cnt_ref, pos_ref, off_ref, carry_ref):
    i = pl.program_id(0)

    @pl.when(i == 0)
    def _():
        carry_ref[...] = jnp.zeros_like(carry_ref)

    offs = []
    racc = jnp.float32(0.0)
    for c in range(_NBK + 1):
        offs.append(racc)
        racc = racc + cnt_ref[0, c]

    rg = lax.broadcasted_iota(jnp.int32, (_GB, _GB), 0)
    cg = lax.broadcasted_iota(jnp.int32, (_GB, _GB), 1)
    trig_strict = (rg > cg).astype(jnp.float32)
    rl = lax.broadcasted_iota(jnp.int32, (_CK, _CK), 0)
    cl = lax.broadcasted_iota(jnp.int32, (_CK, _CK), 1)
    tril_incl = (rl <= cl).astype(jnp.float32)

    bucket = lax.shift_right_logical(dst_ref[...], 13)
    iot = lax.broadcasted_iota(jnp.int32, (1, _CK), 1)
    carr = carry_ref[...]
    cadd = jnp.zeros((1, _CK), jnp.float32)
    pos = jnp.zeros((_GB, _CK), jnp.float32)
    for cc in range(_NBK):
        mask = (bucket == cc).astype(jnp.float32)
        lane_pre = jnp.dot(mask, tril_incl,
                           preferred_element_type=jnp.float32)
        rowtot = lane_pre[:, _CK - 1:_CK]
        row_pre = jnp.dot(trig_strict, rowtot,
                          preferred_element_type=jnp.float32)
        slot = offs[cc] + carr[0, cc] + row_pre + lane_pre - 1.0
        pos += mask * slot
        cadd = cadd + jnp.where(iot == cc, jnp.sum(rowtot), 0.0)
    carry_ref[...] = carr + cadd

    pos_ref[...] = pos.astype(jnp.int32)

    @pl.when(i == pl.num_programs(0) - 1)
    def _():
        iot2 = lax.broadcasted_iota(jnp.int32, (1, _CK), 1)
        ofr = jnp.zeros((1, _CK), jnp.float32)
        for c in range(_NBK + 1):
            ofr = ofr + jnp.where(iot2 == c, offs[c], 0.0)
        off_ref[...] = ofr.astype(jnp.int32)


def _tc_pos(dst2d, counts):
    g = dst2d.shape[0]
    return pl.pallas_call(
        _pos_body,
        grid=(g // _GB,),
        in_specs=[pl.BlockSpec((_GB, _CK), lambda i: (i, 0)),
                  pl.BlockSpec((1, _CK), lambda i: (0, 0))],
        out_specs=[pl.BlockSpec((_GB, _CK), lambda i: (i, 0)),
                   pl.BlockSpec((1, _CK), lambda i: (0, 0))],
        out_shape=[jax.ShapeDtypeStruct((g, _CK), jnp.int32),
                   jax.ShapeDtypeStruct((1, _CK), jnp.int32)],
        scratch_shapes=[pltpu.VMEM((1, _CK), jnp.float32)],
        compiler_params=pltpu.CompilerParams(
            dimension_semantics=("arbitrary",)),
    )(dst2d, counts)


_STG = 321536


def _segsum2_body(m, posv, dstv, offh, hout,
                  acc, shids, shrel, offv, idrow, relrow, mbuf, fbuf, sem):
    cid = lax.axis_index("c")
    sid = lax.axis_index("s")
    rpt = _SP2C // _NS
    nblk = _EB // _CK
    iota16 = lax.iota(jnp.int32, 16)
    pltpu.sync_copy(offh, offv.at[pl.ds(0, 32)])
    zf = jnp.zeros((16,), jnp.float32)
    zi = jnp.zeros((16,), jnp.int32)

    def frow(i, carry):
        fbuf[pl.ds(i * 16, 16)] = zi
        return carry

    lax.fori_loop(0, 128, frow, 0)

    def fill(j, carry):
        p = (sid + j * _NS) * 2048

        @pl.when(p < _STG)
        def _():
            pltpu.sync_copy(fbuf, shrel.at[pl.ds(p, 2048)])
            pltpu.sync_copy(fbuf, shids.at[pl.ds(p, 2048)])

        return carry

    lax.fori_loop(0, (_STG // 2048 + _NS - 1) // _NS, fill, 0)
    plsc.subcore_barrier()

    def scat(j, carry):
        b = sid + j * _NS

        @pl.when(b < nblk)
        def _():
            r0 = b * _CK
            i0 = pltpu.async_copy(posv.at[pl.ds(r0, _CK)], idrow, sem)
            i1 = pltpu.async_copy(dstv.at[pl.ds(r0, _CK)], relrow, sem)
            i0.wait()
            i1.wait()
            for v in range(_CK // 16):
                pv = idrow[pl.ds(v * 16, 16)]
                dv = relrow[pl.ds(v * 16, 16)]
                own = lax.shift_right_logical(dv, 13) % 2 == cid
                pv = jnp.minimum(jnp.maximum(pv, 0), _EB - 1)
                idrow[pl.ds(v * 16, 16)] = jnp.where(own, pv, _EB)
                relrow[pl.ds(v * 16, 16)] = dv + 1
                fbuf[pl.ds(v * 16, 16)] = (r0 + v * 16) + iota16
            pltpu.sync_copy(fbuf.at[pl.ds(0, _CK)], shids.at[idrow],
                            add=True)
            pltpu.sync_copy(relrow, shrel.at[idrow], add=True)

        return carry

    lax.fori_loop(0, (nblk + _NS - 1) // _NS, scat, 0)
    plsc.subcore_barrier()

    def chunk_loop(kc, carry):
        t = cid + kc * _NC
        lo = t * _SP2C
        o0 = offv[pl.ds(t, 16)][0]
        o1 = offv[pl.ds(t + 1, 16)][0]
        b0 = jnp.minimum(jnp.maximum(lax.shift_right_logical(o0, 7), 0),
                         nblk)
        b1 = jnp.minimum(jnp.maximum(lax.shift_right_logical(o1 + 127, 7),
                                     0), nblk)

        def zrow(i, zcarry):
            for v in range(_D // 16):
                mbuf[i, pl.ds(v * 16, 16)] = zf
            return zcarry

        lax.fori_loop(0, _CK, zrow, 0)
        for r in range(rpt // _CK):
            pltpu.sync_copy(mbuf, acc.at[pl.ds(sid * rpt + r * _CK, _CK)])
        plsc.subcore_barrier()

        def gat(j, gcarry):
            b = b0 + sid + j * _NS

            @pl.when(b < b1)
            def _():
                r0 = b * _CK
                pltpu.sync_copy(shids.at[pl.ds(r0, _CK)], idrow)
                pltpu.sync_copy(shrel.at[pl.ds(r0, _CK)], relrow)
                for v in range(_D // 16):
                    rr = relrow[pl.ds(v * 16, 16)] - (lo + 1)
                    ok = (rr >= 0) & (rr < _SP2C)
                    relrow[pl.ds(v * 16, 16)] = jnp.where(
                        ok, rr, jnp.full((16,), _SP2C, jnp.int32))
                    iv = idrow[pl.ds(v * 16, 16)]
                    idrow[pl.ds(v * 16, 16)] = jnp.minimum(
                        jnp.maximum(iv, 0), _EB - 1)
                pltpu.async_copy(m.at[idrow], mbuf, sem).wait()
                pltpu.sync_copy(mbuf, acc.at[relrow], add=True)

            return gcarry

        lax.fori_loop(0, (b1 - b0 + _NS - 1) // _NS, gat, 0)
        plsc.subcore_barrier()
        for r in range(rpt // _CK):
            rr = sid * rpt + r * _CK
            pltpu.sync_copy(acc.at[pl.ds(rr, _CK)], mbuf)
            pltpu.sync_copy(mbuf, hout.at[pl.ds(lo + rr, _CK)])
        plsc.subcore_barrier()
        return carry

    lax.fori_loop(0, _NCH2 // _NC, chunk_loop, 0)


def _segsum2(m, dstv):
    pad = _GPAD * _CK - _EB
    dstp = jnp.concatenate(
        [dstv, jnp.full((pad,), _NBK * _SP2C, jnp.int32)])
    dst2d = dstp.reshape(_GPAD, _CK)
    counts = _tc_hist(dst2d)
    pos2d, off = _tc_pos(dst2d, counts)
    f = pl.kernel(
        _segsum2_body,
        out_type=jax.ShapeDtypeStruct((_SP2, _D), jnp.float32),
        mesh=_mesh(),
        scratch_types=[
            pltpu.VMEM_SHARED((_SP2C + 8, _D), jnp.float32),
            pltpu.VMEM_SHARED((_STG,), jnp.int32),
            pltpu.VMEM_SHARED((_STG,), jnp.int32),
            pltpu.VMEM((64,), jnp.int32),
            pltpu.VMEM((_CK,), jnp.int32),
            pltpu.VMEM((_CK,), jnp.int32),
            pltpu.VMEM((_CK, _D), jnp.float32),
            pltpu.VMEM((2048,), jnp.int32),
            pltpu.SemaphoreType.DMA,
        ],
    )
    return f(m, pos2d.reshape(_GPAD * _CK)[:_EB], dstv,
             off.reshape(_CK)[:32])


_BE = 1280


def _sigmoid(x):
    return 1.0 / (1.0 + jnp.exp(-x))


def _silu(x):
    return x * _sigmoid(x)


def _mlp_body(n_in, n_mult, residual, *refs):
    xs = refs[:n_in]
    i = n_in
    mults = refs[i:i + n_mult]
    i += n_mult
    res = refs[i] if residual else None
    i += 1 if residual else 0
    gw1, gb1, gw2, gb2, ow1, ob1, ow2, ob2, out = refs[i:i + 9]

    xcat = jnp.concatenate([x[...] for x in xs], axis=1)
    ag = jnp.dot(xcat, gw1[...], preferred_element_type=jnp.float32)
    ao = jnp.dot(xcat, ow1[...], preferred_element_type=jnp.float32)
    hg = _silu(ag + gb1[...])
    ho = _silu(ao + ob1[...])
    g = _sigmoid(jnp.dot(hg, gw2[...], preferred_element_type=jnp.float32)
                 + gb2[...])
    o = _silu(jnp.dot(ho, ow2[...], preferred_element_type=jnp.float32)
              + ob2[...])
    y = o * g
    for mr in mults:
        y = y * mr[...]
    if residual:
        y = y + res[...]
    out[...] = y


def _mlp(xs, mults, res, p):
    e = xs[0].shape[0]
    n_in = len(xs)
    grid = (e // _BE,)
    row = pl.BlockSpec((_BE, _D), lambda i: (i, 0))
    w1s = pl.BlockSpec((n_in * _D, _H), lambda i: (0, 0))
    b1s = pl.BlockSpec((1, _H), lambda i: (0, 0))
    w2s = pl.BlockSpec((_H, _D), lambda i: (0, 0))
    b2s = pl.BlockSpec((1, _D), lambda i: (0, 0))
    n_row = n_in + len(mults) + (1 if res is not None else 0)
    in_specs = [row] * n_row + [w1s, b1s, w2s, b2s, w1s, b1s, w2s, b2s]
    gw1 = p['gw1']
    ow1 = p['ow1']
    args = ([*xs, *mults] + ([res] if res is not None else [])
            + [gw1, p['gb1'].reshape(1, _H), p['gw2'], p['gb2'].reshape(1, _D),
               ow1, p['ob1'].reshape(1, _H), p['ow2'], p['ob2'].reshape(1, _D)])
    return pl.pallas_call(
        functools.partial(_mlp_body, n_in, len(mults), res is not None),
        grid=grid,
        in_specs=in_specs,
        out_specs=row,
        out_shape=jax.ShapeDtypeStruct((e, _D), jnp.float32),
        compiler_params=pltpu.CompilerParams(
            dimension_semantics=("arbitrary",)),
    )(*args)


def _lin_body(nh, *refs):
    feat = refs[0]
    hs = refs[1:1 + nh]
    w, b, out = refs[1 + nh:1 + nh + 3]
    h = hs[0][...]
    for k in range(1, nh):
        h = h + hs[k][...]
    out[...] = (feat[...]
                + jnp.dot(h, w[...], preferred_element_type=jnp.float32)
                + b[...])


def _lin(feat, hs, w, b, be):
    e = feat.shape[0]
    grid = (e // be,)
    row = pl.BlockSpec((be, _D), lambda i: (i, 0))
    ws = pl.BlockSpec((_D, _D), lambda i: (0, 0))
    bs = pl.BlockSpec((1, _D), lambda i: (0, 0))
    return pl.pallas_call(
        functools.partial(_lin_body, len(hs)),
        grid=grid,
        in_specs=[row] * (1 + len(hs)) + [ws, bs],
        out_specs=row,
        out_shape=jax.ShapeDtypeStruct((e, _D), jnp.float32),
        compiler_params=pltpu.CompilerParams(
            dimension_semantics=("arbitrary",)),
    )(feat, *hs, w, b.reshape(1, _D))


def kernel(atom_feat, bond_feat, angle_feat, atom_edge_index, bond_edge_index,
           angle_index, atom_bond_weight, bond_node_weight, params):
    src_a = atom_edge_index[0]
    dst_a = atom_edge_index[1]
    src_b = bond_edge_index[0]
    dst_b = bond_edge_index[1]
    vertex = angle_index[:, 1]

    g1s, g1d = _gather2(atom_feat, src_a, dst_a)
    bfs, bfd = _gather2(bond_feat, src_b, dst_b)
    ws, wd = _gather2(bond_node_weight, src_b, dst_b)
    m1 = _mlp([g1s, g1d, bond_feat], [atom_bond_weight], None,
              params['atom_conv'])
    hparts = _segsum1(m1, dst_a)
    atom_out = _lin(atom_feat, [hparts[0, :_NA], hparts[1, :_NA]],
                    params['atom_lin']['w'], params['atom_lin']['b'], 1000)

    vf = _gather1(atom_out, vertex)
    m2 = _mlp([bfs, bfd, angle_feat, vf], [ws, wd], None, params['bond_conv'])
    h2 = _segsum2(m2, dst_b)
    bond_out = _lin(bond_feat, [h2[:_EA]],
                    params['bond_lin']['w'], params['bond_lin']['b'], _BE)

    g3s, g3d = _gather2(bond_out, src_b, dst_b)
    angle_out = _mlp([g3s, g3d, angle_feat, vf], [], angle_feat,
                     params['angle_update'])

    return (atom_out, bond_out, angle_out)
